# baseline (speedup 1.0000x reference)
import functools
import math

import jax
import jax.numpy as jnp
from jax import lax
from jax.experimental import pallas as pl
from jax.experimental.pallas import tpu as pltpu
from jax.experimental.pallas import tpu_sc as plsc

_BETA = 0.5
_LM1 = math.log(0.01)
_LM2 = math.log(400.0)
_NBINS = 100
_SCALE = _NBINS / (_LM2 - _LM1)
_SCALE16 = 16.0 * _SCALE
_OFFS16 = 16.0 * (1.0 - _LM1 * _SCALE)
_HBINS = _NBINS + 2
_LANES = 16

_N = 4096 * 4096
_NC = 2
_NS = 16
_NW = _NC * _NS
_ROWS_W = 4096 // _NW
_SLAB_R = 8
_SLAB_C = 2048
_SLAB_ELEMS = _SLAB_R * _SLAB_C
_NSLAB = _ROWS_W * 4096 // _SLAB_ELEMS
_NBUF = 2
_UNROLL = 4
_OUTW = 112

_ROWS_PER_STEP = 128
_GRID = 4096 // _ROWS_PER_STEP


def _sc_hist_body(pred_hbm, act_hbm, hp_hbm, ha_hbm, mse_hbm,
                  bp0, bp1, ba0, ba1, histp, hista, outp, outa, outm,
                  sp0, sp1, sa0, sa1):
    wid = lax.axis_index("s") * _NC + lax.axis_index("c")
    base_row = wid * _ROWS_W

    zeros16 = jnp.zeros((_LANES,), jnp.float32)
    hsize = _HBINS * _LANES

    def zero_body(i, _):
        histp[pl.ds(i * _LANES, _LANES)] = zeros16
        hista[pl.ds(i * _LANES, _LANES)] = zeros16
        return 0

    lax.fori_loop(0, _HBINS * _UNROLL, zero_body, 0)

    bufs_p = [bp0, bp1]
    bufs_a = [ba0, ba1]
    sems_p = [sp0, sp1]
    sems_a = [sa0, sa1]

    def slab_slice(sid):
        r0 = base_row + (sid // 2) * _SLAB_R
        c0 = (sid % 2) * _SLAB_C
        return (pl.ds(r0, _SLAB_R), pl.ds(c0, _SLAB_C))

    for b in range(_NBUF):
        sl = slab_slice(b)
        pltpu.async_copy(pred_hbm.at[sl], bufs_p[b], sems_p[b])
        pltpu.async_copy(act_hbm.at[sl], bufs_a[b], sems_a[b])

    lane = lax.iota(jnp.int32, _LANES)
    lane_u = [lane + u * hsize for u in range(_UNROLL)]
    ones16 = jnp.ones((_LANES,), jnp.float32)
    scale16 = jnp.float32(_SCALE16)
    offs16 = jnp.float32(_OFFS16)
    topslot = jnp.uint32((_HBINS - 1) * _LANES + _LANES - 1)
    binmask = jnp.int32(-_LANES)

    def process(bufp, bufa, accs):
        nblk_row = _SLAB_C // _LANES

        def body(j, accs):
            new = []
            for u in range(_UNROLL):
                k = j * _UNROLL + u
                r = k // nblk_row
                c = (k % nblk_row) * _LANES
                xp = bufp[r, pl.ds(c, _LANES)]
                xa = bufa[r, pl.ds(c, _LANES)]
                tp = xp * scale16 + offs16
                ta = xa * scale16 + offs16
                up = lax.bitcast_convert_type(tp.astype(jnp.int32),
                                              jnp.uint32)
                ua = lax.bitcast_convert_type(ta.astype(jnp.int32),
                                              jnp.uint32)
                ip = lax.bitcast_convert_type(jnp.minimum(up, topslot),
                                              jnp.int32) & binmask
                ia = lax.bitcast_convert_type(jnp.minimum(ua, topslot),
                                              jnp.int32) & binmask
                plsc.addupdate_scatter(histp, [ip + lane_u[u]], ones16)
                plsc.addupdate_scatter(hista, [ia + lane_u[u]], ones16)
                d = xp - xa
                new.append(accs[u] + d * d)
            return tuple(new)

        return plsc.parallel_loop(
            0, _SLAB_ELEMS // (_UNROLL * _LANES), unroll=8, carry=accs)(body)

    dummy = (pl.ds(0, _SLAB_R), pl.ds(0, _SLAB_C))

    def outer(g, accs):
        for b in range(_NBUF):
            cid = g * _NBUF + b
            pltpu.make_async_copy(
                pred_hbm.at[dummy], bufs_p[b], sems_p[b]).wait()
            pltpu.make_async_copy(
                act_hbm.at[dummy], bufs_a[b], sems_a[b]).wait()
            accs = process(bufs_p[b], bufs_a[b], accs)
            nxt = cid + _NBUF

            @pl.when(nxt < _NSLAB)
            def _():
                sl = slab_slice(nxt)
                pltpu.async_copy(pred_hbm.at[sl], bufs_p[b], sems_p[b])
                pltpu.async_copy(act_hbm.at[sl], bufs_a[b], sems_a[b])
        return accs

    init_accs = tuple(jnp.zeros((_LANES,), jnp.float32)
                      for _ in range(_UNROLL))
    accs = lax.fori_loop(0, _NSLAB // _NBUF, outer, init_accs)
    acc = accs[0]
    for u in range(1, _UNROLL):
        acc = acc + accs[u]
    outm[...] = acc
    pltpu.sync_copy(outm, mse_hbm.at[wid])

    def merge_body(i, _):
        sl = pl.ds(i * _LANES, _LANES)
        accp = histp[sl]
        acca = hista[sl]
        for u in range(1, _UNROLL):
            slu = pl.ds(u * hsize + i * _LANES, _LANES)
            accp = accp + histp[slu]
            acca = acca + hista[slu]
        outp[sl] = accp
        outa[sl] = acca
        return 0

    lax.fori_loop(0, _HBINS, merge_body, 0)

    pltpu.sync_copy(outp, hp_hbm.at[wid])
    pltpu.sync_copy(outa, ha_hbm.at[wid])


def _sc_histograms(pred_flat, act_flat):
    mesh = plsc.VectorSubcoreMesh(core_axis_name="c", subcore_axis_name="s")
    return pl.kernel(
        _sc_hist_body,
        out_type=(
            jax.ShapeDtypeStruct((_NW, _HBINS * _LANES), jnp.float32),
            jax.ShapeDtypeStruct((_NW, _HBINS * _LANES), jnp.float32),
            jax.ShapeDtypeStruct((_NW, _LANES), jnp.float32),
        ),
        mesh=mesh,
        compiler_params=pltpu.CompilerParams(
            needs_layout_passes=False, use_tc_tiling_on_sc=True),
        scratch_types=[
            pltpu.VMEM((_SLAB_R, _SLAB_C), jnp.float32),
            pltpu.VMEM((_SLAB_R, _SLAB_C), jnp.float32),
            pltpu.VMEM((_SLAB_R, _SLAB_C), jnp.float32),
            pltpu.VMEM((_SLAB_R, _SLAB_C), jnp.float32),
            pltpu.VMEM((_UNROLL * _HBINS * _LANES,), jnp.float32),
            pltpu.VMEM((_UNROLL * _HBINS * _LANES,), jnp.float32),
            pltpu.VMEM((_HBINS * _LANES,), jnp.float32),
            pltpu.VMEM((_HBINS * _LANES,), jnp.float32),
            pltpu.VMEM((_LANES,), jnp.float32),
            pltpu.SemaphoreType.DMA,
            pltpu.SemaphoreType.DMA,
            pltpu.SemaphoreType.DMA,
            pltpu.SemaphoreType.DMA,
        ],
    )(pred_flat, act_flat)


def _tc_body(hp_ref, ha_ref, mse_ref, out_ref):
    mse = jnp.sum(mse_ref[...]) / jnp.float32(_N)
    flat_bin = lax.broadcasted_iota(
        jnp.int32, (_HBINS * _LANES, _OUTW), 0) // _LANES
    col = lax.broadcasted_iota(jnp.int32, (_HBINS * _LANES, _OUTW), 1)
    group = (flat_bin == col + 1).astype(jnp.float32)
    hp = jnp.dot(jnp.sum(hp_ref[...], axis=0, keepdims=True), group,
                 preferred_element_type=jnp.float32)
    ha = jnp.dot(jnp.sum(ha_ref[...], axis=0, keepdims=True), group,
                 preferred_element_type=jnp.float32)
    valid = lax.broadcasted_iota(jnp.int32, (1, _OUTW), 1) < _NBINS
    cp = jnp.where(valid, hp + 1e-5, 0.0)
    cq = jnp.where(valid, ha + 1e-5, 0.0)
    p = cp / jnp.sum(cp)
    q = cq / jnp.sum(cq)
    log_p = jnp.log(jnp.where(valid, p, 1.0))
    log_q = jnp.log(jnp.where(valid, q, 1.0))
    kld = jnp.sum(jnp.where(valid, q * (log_q - log_p), 0.0)) / _NBINS
    out_ref[...] = jnp.full((1, 1), mse + _BETA * kld, jnp.float32)


def _tc_combine(hp, ha, msep):
    return pl.pallas_call(
        _tc_body,
        out_shape=jax.ShapeDtypeStruct((1, 1), jnp.float32),
    )(hp, ha, msep)


@jax.jit
def kernel(pred, actual):
    hp, ha, msep = _sc_histograms(pred, actual)
    out = _tc_combine(hp, ha, msep)
    return out[0, 0]


# speedup vs baseline: 1.4829x; 1.4829x over previous
import functools
import math

import jax
import jax.numpy as jnp
from jax import lax
from jax.experimental import pallas as pl
from jax.experimental.pallas import tpu as pltpu
from jax.experimental.pallas import tpu_sc as plsc

_BETA = 0.5
_LM1 = math.log(0.01)
_LM2 = math.log(400.0)
_NBINS = 100
_SCALE = _NBINS / (_LM2 - _LM1)
_SCALE16 = 16.0 * _SCALE
_OFFS16 = 16.0 * (1.0 - _LM1 * _SCALE)
_HBINS = _NBINS + 2
_LANES = 16

_N = 4096 * 4096
_NC = 2
_NS = 16
_NW = _NC * _NS
_ROWS_W = 4096 // _NW
_SLAB_R = 8
_SLAB_C = 2048
_SLAB_ELEMS = _SLAB_R * _SLAB_C
_NSLAB = _ROWS_W * 4096 // _SLAB_ELEMS
_NBUF = 2
_UNROLL = 4
_OUTW = 112

_ROWS_PER_STEP = 128
_GRID = 4096 // _ROWS_PER_STEP


def _sc_hist_body(pred_hbm, act_hbm, hp_hbm, ha_hbm,
                  bp0, bp1, ba0, ba1, histp, hista, outp, outa,
                  sp0, sp1, sa0, sa1):
    wid = lax.axis_index("s") * _NC + lax.axis_index("c")
    base_row = wid * _ROWS_W

    zeros16 = jnp.zeros((_LANES,), jnp.float32)
    hsize = _HBINS * _LANES

    def zero_body(i, _):
        histp[pl.ds(i * _LANES, _LANES)] = zeros16
        hista[pl.ds(i * _LANES, _LANES)] = zeros16
        return 0

    lax.fori_loop(0, _HBINS * _UNROLL, zero_body, 0)

    bufs_p = [bp0, bp1]
    bufs_a = [ba0, ba1]
    sems_p = [sp0, sp1]
    sems_a = [sa0, sa1]

    def slab_slice(sid):
        r0 = base_row + (sid // 2) * _SLAB_R
        c0 = (sid % 2) * _SLAB_C
        return (pl.ds(r0, _SLAB_R), pl.ds(c0, _SLAB_C))

    for b in range(_NBUF):
        sl = slab_slice(b)
        pltpu.async_copy(pred_hbm.at[sl], bufs_p[b], sems_p[b])
        pltpu.async_copy(act_hbm.at[sl], bufs_a[b], sems_a[b])

    lane = lax.iota(jnp.int32, _LANES)
    lane_u = [lane + u * hsize for u in range(_UNROLL)]
    ones16 = jnp.ones((_LANES,), jnp.float32)
    scale16 = jnp.float32(_SCALE16)
    offs16 = jnp.float32(_OFFS16)
    topslot = jnp.uint32((_HBINS - 1) * _LANES + _LANES - 1)
    binmask = jnp.int32(-_LANES)

    def process(bufp, bufa):
        nblk_row = _SLAB_C // _LANES

        def body(j):
            for u in range(_UNROLL):
                k = j * _UNROLL + u
                r = k // nblk_row
                c = (k % nblk_row) * _LANES
                xp = bufp[r, pl.ds(c, _LANES)]
                xa = bufa[r, pl.ds(c, _LANES)]
                tp = xp * scale16 + offs16
                ta = xa * scale16 + offs16
                up = lax.bitcast_convert_type(tp.astype(jnp.int32),
                                              jnp.uint32)
                ua = lax.bitcast_convert_type(ta.astype(jnp.int32),
                                              jnp.uint32)
                ip = lax.bitcast_convert_type(jnp.minimum(up, topslot),
                                              jnp.int32) & binmask
                ia = lax.bitcast_convert_type(jnp.minimum(ua, topslot),
                                              jnp.int32) & binmask
                plsc.addupdate_scatter(histp, [ip + lane_u[u]], ones16)
                plsc.addupdate_scatter(hista, [ia + lane_u[u]], ones16)

        plsc.parallel_loop(
            0, _SLAB_ELEMS // (_UNROLL * _LANES), unroll=4)(body)

    dummy = (pl.ds(0, _SLAB_R), pl.ds(0, _SLAB_C))

    def outer(g, _):
        for b in range(_NBUF):
            cid = g * _NBUF + b
            pltpu.make_async_copy(
                pred_hbm.at[dummy], bufs_p[b], sems_p[b]).wait()
            pltpu.make_async_copy(
                act_hbm.at[dummy], bufs_a[b], sems_a[b]).wait()
            process(bufs_p[b], bufs_a[b])
            nxt = cid + _NBUF

            @pl.when(nxt < _NSLAB)
            def _():
                sl = slab_slice(nxt)
                pltpu.async_copy(pred_hbm.at[sl], bufs_p[b], sems_p[b])
                pltpu.async_copy(act_hbm.at[sl], bufs_a[b], sems_a[b])
        return 0

    lax.fori_loop(0, _NSLAB // _NBUF, outer, 0)

    def merge_body(i, _):
        sl = pl.ds(i * _LANES, _LANES)
        accp = histp[sl]
        acca = hista[sl]
        for u in range(1, _UNROLL):
            slu = pl.ds(u * hsize + i * _LANES, _LANES)
            accp = accp + histp[slu]
            acca = acca + hista[slu]
        outp[sl] = accp
        outa[sl] = acca
        return 0

    lax.fori_loop(0, _HBINS, merge_body, 0)

    pltpu.sync_copy(outp, hp_hbm.at[wid])
    pltpu.sync_copy(outa, ha_hbm.at[wid])


def _sc_histograms(pred_flat, act_flat):
    mesh = plsc.VectorSubcoreMesh(core_axis_name="c", subcore_axis_name="s")
    return pl.kernel(
        _sc_hist_body,
        out_type=(
            jax.ShapeDtypeStruct((_NW, _HBINS * _LANES), jnp.float32),
            jax.ShapeDtypeStruct((_NW, _HBINS * _LANES), jnp.float32),
        ),
        mesh=mesh,
        compiler_params=pltpu.CompilerParams(
            needs_layout_passes=False, use_tc_tiling_on_sc=True),
        scratch_types=[
            pltpu.VMEM((_SLAB_R, _SLAB_C), jnp.float32),
            pltpu.VMEM((_SLAB_R, _SLAB_C), jnp.float32),
            pltpu.VMEM((_SLAB_R, _SLAB_C), jnp.float32),
            pltpu.VMEM((_SLAB_R, _SLAB_C), jnp.float32),
            pltpu.VMEM((_UNROLL * _HBINS * _LANES,), jnp.float32),
            pltpu.VMEM((_UNROLL * _HBINS * _LANES,), jnp.float32),
            pltpu.VMEM((_HBINS * _LANES,), jnp.float32),
            pltpu.VMEM((_HBINS * _LANES,), jnp.float32),
            pltpu.SemaphoreType.DMA,
            pltpu.SemaphoreType.DMA,
            pltpu.SemaphoreType.DMA,
            pltpu.SemaphoreType.DMA,
        ],
    )(pred_flat, act_flat)


def _tc_mse_body(p_ref, a_ref, out_ref, acc_ref):
    i = pl.program_id(0)

    @pl.when(i == 0)
    def _():
        acc_ref[0] = 0.0

    d = p_ref[...] - a_ref[...]
    acc_ref[0] += jnp.sum(d * d)

    @pl.when(i == _GRID - 1)
    def _():
        out_ref[...] = jnp.full((1, 1), acc_ref[0], jnp.float32)


def _tc_mse(pred, actual):
    return pl.pallas_call(
        _tc_mse_body,
        grid=(_GRID,),
        in_specs=[
            pl.BlockSpec((_ROWS_PER_STEP, 4096), lambda i: (i, 0)),
            pl.BlockSpec((_ROWS_PER_STEP, 4096), lambda i: (i, 0)),
        ],
        out_specs=pl.BlockSpec((1, 1), lambda i: (0, 0)),
        out_shape=jax.ShapeDtypeStruct((1, 1), jnp.float32),
        scratch_shapes=[pltpu.SMEM((1,), jnp.float32)],
    )(pred, actual)


def _tc_body(hp_ref, ha_ref, mse_ref, out_ref):
    mse = mse_ref[0, 0] / jnp.float32(_N)
    flat_bin = lax.broadcasted_iota(
        jnp.int32, (_HBINS * _LANES, _OUTW), 0) // _LANES
    col = lax.broadcasted_iota(jnp.int32, (_HBINS * _LANES, _OUTW), 1)
    group = (flat_bin == col + 1).astype(jnp.float32)
    hp = jnp.dot(jnp.sum(hp_ref[...], axis=0, keepdims=True), group,
                 preferred_element_type=jnp.float32)
    ha = jnp.dot(jnp.sum(ha_ref[...], axis=0, keepdims=True), group,
                 preferred_element_type=jnp.float32)
    valid = lax.broadcasted_iota(jnp.int32, (1, _OUTW), 1) < _NBINS
    cp = jnp.where(valid, hp + 1e-5, 0.0)
    cq = jnp.where(valid, ha + 1e-5, 0.0)
    p = cp / jnp.sum(cp)
    q = cq / jnp.sum(cq)
    log_p = jnp.log(jnp.where(valid, p, 1.0))
    log_q = jnp.log(jnp.where(valid, q, 1.0))
    kld = jnp.sum(jnp.where(valid, q * (log_q - log_p), 0.0)) / _NBINS
    out_ref[...] = jnp.full((1, 1), mse + _BETA * kld, jnp.float32)


def _tc_combine(hp, ha, msep):
    return pl.pallas_call(
        _tc_body,
        out_shape=jax.ShapeDtypeStruct((1, 1), jnp.float32),
    )(hp, ha, msep)


@jax.jit
def kernel(pred, actual):
    msesum = _tc_mse(pred, actual)
    hp, ha = _sc_histograms(pred, actual)
    out = _tc_combine(hp, ha, msesum)
    return out[0, 0]
